# baseline (speedup 1.0000x reference)
import jax
import jax.numpy as jnp
from jax.experimental import pallas as pl
from jax.experimental.pallas import tpu as pltpu
from jax.experimental.pallas import tpu_sc as plsc

B, N, DE = 32, 576, 768
H, D = 8, 64
K = 1024
DM = 64
BN = B * N
HBN = H * BN

ROWS = 256
RB = 64
KC = 128
GW = 128
MROWS = 512
CHUNKS = 4


def _prep_body(cb_ref, cbsq_ref, cb2_ref):
    cb = cb_ref[0]
    cbsq_ref[0] = jnp.sum(cb * cb, axis=1)[None, :]
    cb2_ref[0] = cb + cb


def _vq_body(x_ref, rp_ref, cb2_ref, cbsq_ref, idx_ref):
    xv = x_ref[...]
    xp = jax.lax.dot_general(
        xv, rp_ref[...], (((1,), (0,)), ((), ())),
        preferred_element_type=jnp.float32)
    lane = jax.lax.broadcasted_iota(jnp.int32, (RB, KC), 1).astype(jnp.float32)
    for h in range(H):
        xph = xp[:, h * D:(h + 1) * D]
        x_sq = jnp.sum(xph * xph, axis=1, keepdims=True)
        cross2 = jax.lax.dot_general(
            xph, cb2_ref[h], (((1,), (1,)), ((), ())),
            preferred_element_type=jnp.float32)
        cq = cbsq_ref[h]
        for r0 in range(0, ROWS, RB):
            xs = x_sq[r0 : r0 + RB]
            acc_v = (xs - cross2[r0 : r0 + RB, 0:KC]) + cq[:, 0:KC]
            acc_c = jnp.zeros((RB, KC), jnp.float32)
            for kc in range(KC, K, KC):
                d = ((xs - cross2[r0 : r0 + RB, kc : kc + KC])
                     + cq[:, kc : kc + KC])
                better = d < acc_v
                acc_v = jnp.where(better, d, acc_v)
                acc_c = jnp.where(better, float(kc // KC), acc_c)
            m = jnp.min(acc_v, axis=1, keepdims=True)
            cand = jnp.where(acc_v == m, acc_c * float(KC) + lane, float(K))
            best = jnp.min(cand, axis=1, keepdims=True)
            idx_ref[r0 : r0 + RB, h : h + 1] = best.astype(jnp.int32) + h * K


def _mean_body(g_ref, o_ref):
    o_ref[...] = jnp.sum(g_ref[...], axis=1) * (1.0 / H)


def _make_indices(x2, rp_all, cb2, cbsq):
    nb = x2.shape[0]
    return pl.pallas_call(
        _vq_body,
        grid=(nb // ROWS,),
        in_specs=[
            pl.BlockSpec((ROWS, DE), lambda i: (i, 0)),
            pl.BlockSpec((DE, H * D), lambda i: (0, 0)),
            pl.BlockSpec((H, K, D), lambda i: (0, 0, 0)),
            pl.BlockSpec((H, 1, K), lambda i: (0, 0, 0)),
        ],
        out_specs=pl.BlockSpec((ROWS, H), lambda i: (i, 0)),
        out_shape=jax.ShapeDtypeStruct((nb, H), jnp.int32),
    )(x2, rp_all, cb2, cbsq)


def _mean(g3):
    nb = g3.shape[0]
    return pl.pallas_call(
        _mean_body,
        grid=(nb // MROWS,),
        in_specs=[pl.BlockSpec((MROWS, H, DM), lambda i: (i, 0, 0))],
        out_specs=pl.BlockSpec((MROWS, DM), lambda i: (i, 0)),
        out_shape=jax.ShapeDtypeStruct((nb, DM), jnp.float32),
    )(g3)


def _sc_gather(values_flat, idx_flat):
    n = idx_flat.shape[1]
    mesh = plsc.VectorSubcoreMesh(core_axis_name="core",
                                  subcore_axis_name="subcore")

    @pl.kernel(out_type=jax.ShapeDtypeStruct((n, DM), jnp.float32),
               mesh=mesh, scratch_types=[],
               compiler_params=pltpu.CompilerParams(use_tc_tiling_on_sc=False))
    def k(tbl_hbm, i_hbm, o_hbm):
        def body(i_vmem, o_vmem):
            pltpu.sync_copy(tbl_hbm.at[i_vmem.at[0]], o_vmem)

        pltpu.emit_pipeline(
            body,
            grid=(n // GW,),
            in_specs=[pl.BlockSpec((1, GW), lambda i: (0, i))],
            out_specs=[pl.BlockSpec((GW, DM), lambda i: (i, 0))],
            core_axis_name=("core", "subcore"),
            dimension_semantics=(pltpu.PARALLEL,),
        )(i_hbm, o_hbm)

    return k(values_flat, idx_flat)


def kernel(x, rand_proj, codebook, values):
    cbsq, cb2 = pl.pallas_call(
        _prep_body,
        grid=(H,),
        in_specs=[pl.BlockSpec((1, K, D), lambda h: (h, 0, 0))],
        out_specs=[
            pl.BlockSpec((1, 1, K), lambda h: (h, 0, 0)),
            pl.BlockSpec((1, K, D), lambda h: (h, 0, 0)),
        ],
        out_shape=[
            jax.ShapeDtypeStruct((H, 1, K), jnp.float32),
            jax.ShapeDtypeStruct((H, K, D), jnp.float32),
        ],
    )(codebook)

    x2 = x.reshape(BN, DE)
    rp_all = rand_proj.transpose(1, 0, 2).reshape(DE, H * D)
    values_flat = values.reshape(H * K, DM)

    cb_rows = BN // CHUNKS
    outs = []
    for c in range(CHUNKS):
        xc = jax.lax.slice(x2, (c * cb_rows, 0), ((c + 1) * cb_rows, DE))
        idx = _make_indices(xc, rp_all, cb2, cbsq)
        g = _sc_gather(values_flat, idx.reshape(1, cb_rows * H))
        outs.append(_mean(g.reshape(cb_rows, H, DM)))

    return jnp.concatenate(outs, axis=0).reshape(B, N, DM)


# speedup vs baseline: 1.0295x; 1.0295x over previous
import jax
import jax.numpy as jnp
from jax.experimental import pallas as pl
from jax.experimental.pallas import tpu as pltpu
from jax.experimental.pallas import tpu_sc as plsc

B, N, DE = 32, 576, 768
H, D = 8, 64
K = 1024
DM = 64
BN = B * N
HBN = H * BN

ROWS = 256
RB = 64
KC = 128
GW = 128
MROWS = 512
CHUNKS = 2


def _prep_body(cb_ref, cbsq_ref, cb2_ref):
    cb = cb_ref[0]
    cbsq_ref[0] = jnp.sum(cb * cb, axis=1)[None, :]
    cb2_ref[0] = cb + cb


def _vq_body(x_ref, rp_ref, cb2_ref, cbsq_ref, idx_ref):
    xv = x_ref[...]
    xp = jax.lax.dot_general(
        xv, rp_ref[...], (((1,), (0,)), ((), ())),
        preferred_element_type=jnp.float32)
    lane = jax.lax.broadcasted_iota(jnp.int32, (RB, KC), 1).astype(jnp.float32)
    for h in range(H):
        xph = xp[:, h * D:(h + 1) * D]
        x_sq = jnp.sum(xph * xph, axis=1, keepdims=True)
        cross2 = jax.lax.dot_general(
            xph, cb2_ref[h], (((1,), (1,)), ((), ())),
            preferred_element_type=jnp.float32)
        cq = cbsq_ref[h]
        for r0 in range(0, ROWS, RB):
            xs = x_sq[r0 : r0 + RB]
            acc_v = (xs - cross2[r0 : r0 + RB, 0:KC]) + cq[:, 0:KC]
            acc_c = jnp.zeros((RB, KC), jnp.float32)
            for kc in range(KC, K, KC):
                d = ((xs - cross2[r0 : r0 + RB, kc : kc + KC])
                     + cq[:, kc : kc + KC])
                better = d < acc_v
                acc_v = jnp.where(better, d, acc_v)
                acc_c = jnp.where(better, float(kc // KC), acc_c)
            m = jnp.min(acc_v, axis=1, keepdims=True)
            cand = jnp.where(acc_v == m, acc_c * float(KC) + lane, float(K))
            best = jnp.min(cand, axis=1, keepdims=True)
            idx_ref[r0 : r0 + RB, h : h + 1] = best.astype(jnp.int32) + h * K


def _mean_body(g_ref, o_ref):
    o_ref[...] = jnp.sum(g_ref[...], axis=1) * (1.0 / H)


def _make_indices(x2, rp_all, cb2, cbsq):
    nb = x2.shape[0]
    return pl.pallas_call(
        _vq_body,
        grid=(nb // ROWS,),
        in_specs=[
            pl.BlockSpec((ROWS, DE), lambda i: (i, 0)),
            pl.BlockSpec((DE, H * D), lambda i: (0, 0)),
            pl.BlockSpec((H, K, D), lambda i: (0, 0, 0)),
            pl.BlockSpec((H, 1, K), lambda i: (0, 0, 0)),
        ],
        out_specs=pl.BlockSpec((ROWS, H), lambda i: (i, 0)),
        out_shape=jax.ShapeDtypeStruct((nb, H), jnp.int32),
    )(x2, rp_all, cb2, cbsq)


def _mean(g3):
    nb = g3.shape[0]
    return pl.pallas_call(
        _mean_body,
        grid=(nb // MROWS,),
        in_specs=[pl.BlockSpec((MROWS, H, DM), lambda i: (i, 0, 0))],
        out_specs=pl.BlockSpec((MROWS, DM), lambda i: (i, 0)),
        out_shape=jax.ShapeDtypeStruct((nb, DM), jnp.float32),
    )(g3)


def _sc_gather(values_flat, idx_flat):
    n = idx_flat.shape[1]
    mesh = plsc.VectorSubcoreMesh(core_axis_name="core",
                                  subcore_axis_name="subcore")

    @pl.kernel(out_type=jax.ShapeDtypeStruct((n, DM), jnp.float32),
               mesh=mesh, scratch_types=[],
               compiler_params=pltpu.CompilerParams(use_tc_tiling_on_sc=False))
    def k(tbl_hbm, i_hbm, o_hbm):
        def body(i_vmem, o_vmem):
            pltpu.sync_copy(tbl_hbm.at[i_vmem.at[0]], o_vmem)

        pltpu.emit_pipeline(
            body,
            grid=(n // GW,),
            in_specs=[pl.BlockSpec((1, GW), lambda i: (0, i))],
            out_specs=[pl.BlockSpec((GW, DM), lambda i: (i, 0))],
            core_axis_name=("core", "subcore"),
            dimension_semantics=(pltpu.PARALLEL,),
        )(i_hbm, o_hbm)

    return k(values_flat, idx_flat)


def kernel(x, rand_proj, codebook, values):
    cbsq, cb2 = pl.pallas_call(
        _prep_body,
        grid=(H,),
        in_specs=[pl.BlockSpec((1, K, D), lambda h: (h, 0, 0))],
        out_specs=[
            pl.BlockSpec((1, 1, K), lambda h: (h, 0, 0)),
            pl.BlockSpec((1, K, D), lambda h: (h, 0, 0)),
        ],
        out_shape=[
            jax.ShapeDtypeStruct((H, 1, K), jnp.float32),
            jax.ShapeDtypeStruct((H, K, D), jnp.float32),
        ],
    )(codebook)

    x2 = x.reshape(BN, DE)
    rp_all = rand_proj.transpose(1, 0, 2).reshape(DE, H * D)
    values_flat = values.reshape(H * K, DM)

    cb_rows = BN // CHUNKS
    outs = []
    for c in range(CHUNKS):
        xc = jax.lax.slice(x2, (c * cb_rows, 0), ((c + 1) * cb_rows, DE))
        idx = _make_indices(xc, rp_all, cb2, cbsq)
        g = _sc_gather(values_flat, idx.reshape(1, cb_rows * H))
        outs.append(_mean(g.reshape(cb_rows, H, DM)))

    return jnp.concatenate(outs, axis=0).reshape(B, N, DM)


# speedup vs baseline: 1.0801x; 1.0492x over previous
import jax
import jax.numpy as jnp
from jax.experimental import pallas as pl
from jax.experimental.pallas import tpu as pltpu
from jax.experimental.pallas import tpu_sc as plsc

B, N, DE = 32, 576, 768
H, D = 8, 64
K = 1024
DM = 64
BN = B * N
HBN = H * BN

ROWS = 256
RB = 64
KC = 128
GW = 128
MROWS = 512
CHUNKS = 1


def _prep_body(cb_ref, cbsq_ref, cb2_ref):
    cb = cb_ref[0]
    cbsq_ref[0] = jnp.sum(cb * cb, axis=1)[None, :]
    cb2_ref[0] = cb + cb


def _vq_body(x_ref, rp_ref, cb2_ref, cbsq_ref, idx_ref):
    xv = x_ref[...]
    xp = jax.lax.dot_general(
        xv, rp_ref[...], (((1,), (0,)), ((), ())),
        preferred_element_type=jnp.float32)
    lane = jax.lax.broadcasted_iota(jnp.int32, (RB, KC), 1).astype(jnp.float32)
    for h in range(H):
        xph = xp[:, h * D:(h + 1) * D]
        x_sq = jnp.sum(xph * xph, axis=1, keepdims=True)
        cross2 = jax.lax.dot_general(
            xph, cb2_ref[h], (((1,), (1,)), ((), ())),
            preferred_element_type=jnp.float32)
        cq = cbsq_ref[h]
        for r0 in range(0, ROWS, RB):
            xs = x_sq[r0 : r0 + RB]
            acc_v = (xs - cross2[r0 : r0 + RB, 0:KC]) + cq[:, 0:KC]
            acc_c = jnp.zeros((RB, KC), jnp.float32)
            for kc in range(KC, K, KC):
                d = ((xs - cross2[r0 : r0 + RB, kc : kc + KC])
                     + cq[:, kc : kc + KC])
                better = d < acc_v
                acc_v = jnp.where(better, d, acc_v)
                acc_c = jnp.where(better, float(kc // KC), acc_c)
            m = jnp.min(acc_v, axis=1, keepdims=True)
            cand = jnp.where(acc_v == m, acc_c * float(KC) + lane, float(K))
            best = jnp.min(cand, axis=1, keepdims=True)
            idx_ref[r0 : r0 + RB, h : h + 1] = best.astype(jnp.int32) + h * K


def _mean_body(g_ref, o_ref):
    o_ref[...] = jnp.sum(g_ref[...], axis=1) * (1.0 / H)


def _make_indices(x2, rp_all, cb2, cbsq):
    nb = x2.shape[0]
    return pl.pallas_call(
        _vq_body,
        grid=(nb // ROWS,),
        in_specs=[
            pl.BlockSpec((ROWS, DE), lambda i: (i, 0)),
            pl.BlockSpec((DE, H * D), lambda i: (0, 0)),
            pl.BlockSpec((H, K, D), lambda i: (0, 0, 0)),
            pl.BlockSpec((H, 1, K), lambda i: (0, 0, 0)),
        ],
        out_specs=pl.BlockSpec((ROWS, H), lambda i: (i, 0)),
        out_shape=jax.ShapeDtypeStruct((nb, H), jnp.int32),
    )(x2, rp_all, cb2, cbsq)


def _mean(g3):
    nb = g3.shape[0]
    return pl.pallas_call(
        _mean_body,
        grid=(nb // MROWS,),
        in_specs=[pl.BlockSpec((MROWS, H, DM), lambda i: (i, 0, 0))],
        out_specs=pl.BlockSpec((MROWS, DM), lambda i: (i, 0)),
        out_shape=jax.ShapeDtypeStruct((nb, DM), jnp.float32),
    )(g3)


def _sc_gather(values_flat, idx_flat):
    n = idx_flat.shape[1]
    mesh = plsc.VectorSubcoreMesh(core_axis_name="core",
                                  subcore_axis_name="subcore")

    @pl.kernel(out_type=jax.ShapeDtypeStruct((n, DM), jnp.float32),
               mesh=mesh, scratch_types=[],
               compiler_params=pltpu.CompilerParams(use_tc_tiling_on_sc=False))
    def k(tbl_hbm, i_hbm, o_hbm):
        def body(i_vmem, o_vmem):
            pltpu.sync_copy(tbl_hbm.at[i_vmem.at[0]], o_vmem)

        pltpu.emit_pipeline(
            body,
            grid=(n // GW,),
            in_specs=[pl.BlockSpec((1, GW), lambda i: (0, i))],
            out_specs=[pl.BlockSpec((GW, DM), lambda i: (i, 0))],
            core_axis_name=("core", "subcore"),
            dimension_semantics=(pltpu.PARALLEL,),
        )(i_hbm, o_hbm)

    return k(values_flat, idx_flat)


def kernel(x, rand_proj, codebook, values):
    cbsq, cb2 = pl.pallas_call(
        _prep_body,
        grid=(H,),
        in_specs=[pl.BlockSpec((1, K, D), lambda h: (h, 0, 0))],
        out_specs=[
            pl.BlockSpec((1, 1, K), lambda h: (h, 0, 0)),
            pl.BlockSpec((1, K, D), lambda h: (h, 0, 0)),
        ],
        out_shape=[
            jax.ShapeDtypeStruct((H, 1, K), jnp.float32),
            jax.ShapeDtypeStruct((H, K, D), jnp.float32),
        ],
    )(codebook)

    x2 = x.reshape(BN, DE)
    rp_all = rand_proj.transpose(1, 0, 2).reshape(DE, H * D)
    values_flat = values.reshape(H * K, DM)

    cb_rows = BN // CHUNKS
    outs = []
    for c in range(CHUNKS):
        xc = jax.lax.slice(x2, (c * cb_rows, 0), ((c + 1) * cb_rows, DE))
        idx = _make_indices(xc, rp_all, cb2, cbsq)
        g = _sc_gather(values_flat, idx.reshape(1, cb_rows * H))
        outs.append(_mean(g.reshape(cb_rows, H, DM)))

    return jnp.concatenate(outs, axis=0).reshape(B, N, DM)
